# baseline (speedup 1.0000x reference)
import functools

import jax
import jax.numpy as jnp
from jax import lax
from jax.experimental import pallas as pl
from jax.experimental.pallas import tpu as pltpu
from jax.experimental.pallas import tpu_sc as plsc

N = 10000
E = 320000
D = 128

NC = 2
NS = 16
NW = NC * NS
L = 16

EPW = E // NW
K = 80
NB = EPW // K
NBUF = 4
QB = 4
QE = QB * K
NOUT = -(-NB // (2 * QB))
EPAD = 256
RPT = 624
ZR = 208
TAIL = N - NS * RPT


def _make_sc_kernel():
    mesh = plsc.VectorSubcoreMesh(
        core_axis_name="c", subcore_axis_name="s",
        num_cores=NC, num_subcores=NS)

    @functools.partial(
        pl.kernel,
        out_type=jax.ShapeDtypeStruct((NC, N, D), jnp.float32),
        mesh=mesh,
        scratch_types=[
            [pltpu.VMEM((K, D), jnp.float32) for _ in range(NBUF)],
            [pltpu.VMEM((QE,), jnp.int32) for _ in range(2)],
            [pltpu.VMEM((QE,), jnp.int32) for _ in range(2)],
            [pltpu.VMEM((QE,), jnp.float32) for _ in range(2)],
            [pltpu.VMEM((K,), jnp.int32) for _ in range(NBUF)],
            pltpu.VMEM_SHARED((N, D), jnp.float32),
            [pltpu.SemaphoreType.DMA for _ in range(2)],
            [pltpu.SemaphoreType.DMA for _ in range(NBUF)],
            [pltpu.SemaphoreType.DMA for _ in range(NBUF)],
        ],
    )
    def spmm(x_hbm, col_hbm, row_hbm, w_hbm, z_hbm, out_hbm,
             rows, colb, rowb, wb, ridx, acc, isem, gsem, ssem):
        cid = lax.axis_index("c")
        sid = lax.axis_index("s")
        wid = cid * NS + sid
        base = wid * EPW

        for j in range(RPT // ZR):
            pltpu.sync_copy(z_hbm, acc.at[pl.ds(sid * RPT + j * ZR, ZR)])

        @pl.when(sid == NS - 1)
        def _zero_tail():
            pltpu.sync_copy(z_hbm.at[pl.ds(0, TAIL)],
                            acc.at[pl.ds(NS * RPT, TAIL)])

        plsc.subcore_barrier()

        def fire_bulk(q, buf):
            off = base + q * QE
            pltpu.async_copy(col_hbm.at[pl.ds(off, QE)], colb[buf], isem[buf])
            pltpu.async_copy(row_hbm.at[pl.ds(off, QE)], rowb[buf], isem[buf])
            pltpu.async_copy(w_hbm.at[pl.ds(off, QE)], wb[buf], isem[buf])

        def wait_bulk(buf):
            pltpu.make_async_copy(
                col_hbm.at[pl.ds(0, QE)], colb[buf], isem[buf]).wait()
            pltpu.make_async_copy(
                row_hbm.at[pl.ds(0, QE)], rowb[buf], isem[buf]).wait()
            pltpu.make_async_copy(
                w_hbm.at[pl.ds(0, QE)], wb[buf], isem[buf]).wait()

        def fire_gather(qbuf, qoff, buf):
            idx_ref = colb[qbuf].at[pl.ds(qoff * K, K)]
            pltpu.async_copy(x_hbm.at[idx_ref], rows[buf], gsem[buf])

        def wait_gather(buf):
            pltpu.make_async_copy(
                x_hbm.at[colb[0].at[pl.ds(0, K)]], rows[buf],
                gsem[buf]).wait()

        def fire_scatter(buf):
            pltpu.async_copy(rows[buf], acc.at[ridx[buf]], ssem[buf],
                             add=True)

        def wait_scatter(buf):
            pltpu.make_async_copy(
                rows[buf], acc.at[ridx[buf]], ssem[buf]).wait()

        fire_bulk(0, 0)
        wait_bulk(0)
        fire_gather(0, 0, 0)

        def outer(ob, _):
            b0 = ob * 2 * QB
            for ph in range(2 * QB):
                bb = b0 + ph
                rbuf = ph % NBUF
                qbuf = (ph // QB) % 2

                if ph == 0:
                    @pl.when(bb + QB < NB)
                    def _fire_b1():
                        fire_bulk(ob * 2 + 1, 1)
                elif ph == QB:
                    @pl.when(bb + QB < NB)
                    def _fire_b0():
                        fire_bulk(ob * 2 + 2, 0)
                elif ph == QB - 1:
                    @pl.when(bb + 1 < NB)
                    def _wait_b1():
                        wait_bulk(1)
                elif ph == 2 * QB - 1:
                    @pl.when(bb + 1 < NB)
                    def _wait_b0():
                        wait_bulk(0)

                ngbuf = (ph + 1) % NBUF
                nqbuf = ((ph + 1) // QB) % 2
                nqoff = (ph + 1) % QB

                @pl.when(bb + 1 < NB)
                def _prefetch_gather():
                    @pl.when(bb >= NBUF - 1)
                    def _drain_scatter():
                        wait_scatter(ngbuf)
                    fire_gather(nqbuf, nqoff, ngbuf)

                @pl.when(bb < NB)
                def _process():
                    wait_gather(rbuf)

                    woff = (ph % QB) * K

                    def group(gr, _):
                        gvec = wb[qbuf][pl.ds(woff + gr * L, L)]
                        for e in range(L):
                            idx = gr * L + e
                            wsc = gvec.at[jnp.full((L,), e, jnp.int32)].get(
                                mode="promise_in_bounds")
                            for j in range(D // L):
                                rows[rbuf][idx, pl.ds(j * L, L)] = (
                                    rows[rbuf][idx, pl.ds(j * L, L)] * wsc)
                        return 0
                    lax.fori_loop(0, K // L, group, 0)

                    for i in range(K // L):
                        ridx[rbuf][pl.ds(i * L, L)] = (
                            rowb[qbuf][pl.ds(woff + i * L, L)])

                    fire_scatter(rbuf)
            return 0
        lax.fori_loop(0, NOUT, outer, 0)

        for ph in range(NBUF):
            wait_scatter(ph)

        plsc.subcore_barrier()

        for j in range(RPT // ZR):
            r0 = sid * RPT + j * ZR
            pltpu.sync_copy(acc.at[pl.ds(r0, ZR)],
                            out_hbm.at[cid, pl.ds(r0, ZR)])

        @pl.when(sid == NS - 1)
        def _copy_tail():
            pltpu.sync_copy(acc.at[pl.ds(NS * RPT, TAIL)],
                            out_hbm.at[cid, pl.ds(NS * RPT, TAIL)])

    return spmm


_sc_spmm = _make_sc_kernel()


def _add_body(a_ref, b_ref, o_ref):
    o_ref[...] = a_ref[...] + b_ref[...]


def _combine(p0, p1):
    blk = 1000
    return pl.pallas_call(
        _add_body,
        out_shape=jax.ShapeDtypeStruct((N, D), jnp.float32),
        grid=(N // blk,),
        in_specs=[pl.BlockSpec((blk, D), lambda i: (i, 0)),
                  pl.BlockSpec((blk, D), lambda i: (i, 0))],
        out_specs=pl.BlockSpec((blk, D), lambda i: (i, 0)),
    )(p0, p1)


def kernel(t, x, edge_index, edge_weight):
    row = jnp.pad(edge_index[0].astype(jnp.int32), (0, EPAD))
    col = jnp.pad(edge_index[1].astype(jnp.int32), (0, EPAD))
    w = jnp.pad(edge_weight.astype(jnp.float32), (0, EPAD))
    z = jnp.zeros((ZR, D), jnp.float32)
    partials = _sc_spmm(x.astype(jnp.float32), col, row, w, z)
    return _combine(partials[0], partials[1])
